# Initial kernel scaffold; baseline (speedup 1.0000x reference)
import jax
import jax.numpy as jnp
from jax.experimental import pallas as pl


def kernel(loc, conf, priors, targets):
    raise NotImplementedError("write your pallas kernel here")


# speedup vs baseline: 12.1590x; 12.1590x over previous
import numpy as np
import jax
import jax.numpy as jnp
from jax.experimental import pallas as pl

_VAR0 = 0.1
_VAR1 = 0.2
_ALPHA = 0.5
_GAMMA = 1.5
_BETA = 0.11
_B = float(np.e ** (_GAMMA / _ALPHA) - 1.0)


def _loss_kernel(loc_ref, conf_ref, priors_ref, targets_ref, out_ref):
    num_t = targets_ref.shape[1]
    P = priors_ref.shape[1]
    C = conf_ref.shape[2]
    f32 = jnp.float32

    step = pl.program_id(0)

    @pl.when(step == 0)
    def _init():
        out_ref[...] = jnp.zeros_like(out_ref)

    cx = priors_ref[0:1, :]
    cy = priors_ref[1:2, :]
    pw = priors_ref[2:3, :]
    ph = priors_ref[3:4, :]
    px0 = cx - pw * 0.5
    py0 = cy - ph * 0.5
    px1 = cx + pw * 0.5
    py1 = cy + ph * 0.5

    tgt = targets_ref[0]
    tx0 = tgt[:, 0:1]
    ty0 = tgt[:, 1:2]
    tx1 = tgt[:, 2:3]
    ty1 = tgt[:, 3:4]
    tlab = tgt[:, 4:5]

    iw = jnp.maximum(jnp.minimum(tx1, px1) - jnp.maximum(tx0, px0), 0.0)
    ih = jnp.maximum(jnp.minimum(ty1, py1) - jnp.maximum(ty0, py0), 0.0)
    inter = iw * ih
    area_t = (tx1 - tx0) * (ty1 - ty0)
    area_p = pw * ph
    ov = inter / (area_t + area_p - inter)

    j_iota = jax.lax.broadcasted_iota(jnp.int32, (num_t, P), 0)
    p_iota = jax.lax.broadcasted_iota(jnp.int32, (num_t, P), 1)

    bto = jnp.max(ov, axis=0, keepdims=True)
    bti = jnp.min(jnp.where(ov == bto, j_iota, num_t), axis=0, keepdims=True)

    bpm = jnp.max(ov, axis=1, keepdims=True)
    bpi = jnp.min(jnp.where(ov == bpm, p_iota, P), axis=1, keepdims=True)
    hit = p_iota == bpi
    j_sel = jnp.max(jnp.where(hit, j_iota, -1), axis=0, keepdims=True)
    forced = j_sel >= 0
    ov_f = jnp.where(forced, 2.0, bto)
    idx_f = jnp.where(forced, j_sel, bti)

    oh = idx_f == j_iota
    m0 = jnp.sum(jnp.where(oh, tx0, 0.0), axis=0, keepdims=True)
    m1 = jnp.sum(jnp.where(oh, ty0, 0.0), axis=0, keepdims=True)
    m2 = jnp.sum(jnp.where(oh, tx1, 0.0), axis=0, keepdims=True)
    m3 = jnp.sum(jnp.where(oh, ty1, 0.0), axis=0, keepdims=True)
    lab = jnp.sum(jnp.where(oh, tlab, 0.0), axis=0, keepdims=True)

    pos = ov_f >= 0.5
    ign = jnp.logical_and(ov_f < 0.5, ov_f >= 0.4)
    npos_step = jnp.sum(jnp.where(pos, 1.0, 0.0))

    g0 = ((m0 + m2) * 0.5 - cx) / (_VAR0 * pw)
    g1 = ((m1 + m3) * 0.5 - cy) / (_VAR0 * ph)
    g2 = jnp.log((m2 - m0) / pw) / _VAR1
    g3 = jnp.log((m3 - m1) / ph) / _VAR1

    loc_sum = jnp.float32(0.0)
    for c, g in enumerate((g0, g1, g2, g3)):
        diff = jnp.abs(loc_ref[0, c:c + 1, :] - g)
        small = (_ALPHA / _B) * (_B * diff + 1.0) * jnp.log(_B * diff / _BETA + 1.0) - _ALPHA * diff
        big = _GAMMA * diff + (_GAMMA / _B - _ALPHA * _BETA)
        bl = jnp.where(diff < _BETA, small, big)
        loc_sum = loc_sum + jnp.sum(jnp.where(pos, bl, 0.0))

    posf = jnp.where(pos, 1.0, 0.0)
    ignf = jnp.where(ign, 1.0, 0.0)
    pack = jnp.concatenate([posf, ignf, lab, jnp.zeros((5, P), f32)], axis=0)
    packT = pack.T
    pos_s = packT[:, 0:1] > 0.5
    ign_s = packT[:, 1:2] > 0.5
    lab_s = packT[:, 2:3]

    x = conf_ref[0]
    c_iota = jax.lax.broadcasted_iota(jnp.int32, (P, C), 1).astype(f32)
    hot = c_iota == lab_s
    t = jnp.where(jnp.logical_and(hot, pos_s), 1.0, 0.0)
    bce = jnp.maximum(x, 0.0) - x * t + jnp.log1p(jnp.exp(-jnp.abs(x)))
    pred = jax.nn.sigmoid(x)
    l = bce * jnp.square(pred - t)
    l = jnp.where(jnp.logical_and(hot, ign_s), 0.0, l)
    conf_sum = jnp.sum(l)

    k_iota = jax.lax.broadcasted_iota(jnp.int32, (1, 128), 1)
    upd = jnp.where(k_iota == 0, loc_sum,
                    jnp.where(k_iota == 1, conf_sum,
                              jnp.where(k_iota == 2, npos_step, 0.0)))
    out_ref[...] += upd


def kernel(loc, conf, priors, targets):
    num, P, C = conf.shape
    n_obj = targets.shape[1]
    loc_t = jnp.transpose(loc, (0, 2, 1))
    priors_t = jnp.transpose(priors)
    out = pl.pallas_call(
        _loss_kernel,
        grid=(num,),
        in_specs=[
            pl.BlockSpec((1, 4, P), lambda i: (i, 0, 0)),
            pl.BlockSpec((1, P, C), lambda i: (i, 0, 0)),
            pl.BlockSpec((4, P), lambda i: (0, 0)),
            pl.BlockSpec((1, n_obj, 5), lambda i: (i, 0, 0)),
        ],
        out_specs=pl.BlockSpec((1, 128), lambda i: (0, 0)),
        out_shape=jax.ShapeDtypeStruct((1, 128), jnp.float32),
    )(loc_t, conf, priors_t, targets)
    npos = out[0, 2]
    denom = jnp.maximum(npos, 1.0)
    return jnp.stack([out[0, 0] / denom, out[0, 1] / denom])
